# Initial kernel scaffold; baseline (speedup 1.0000x reference)
import jax
import jax.numpy as jnp
from jax.experimental import pallas as pl


def kernel(gram_param, idx_X, idx_Y):
    raise NotImplementedError("write your pallas kernel here")


# speedup vs baseline: 2419.6655x; 2419.6655x over previous
import functools

import jax
import jax.numpy as jnp
from jax import lax
from jax.experimental import pallas as pl
from jax.experimental.pallas import tpu as pltpu
from jax.experimental.pallas import tpu_sc as plsc

V_PAD = 1024
N_X = 16384
N_Y = 1024

_NC, _NS = 2, 16
_NW = _NC * _NS
_BPW = N_X // _NW
_CHUNK = 64
_NCHUNK = _BPW // _CHUNK


def _tc_table_body(gp_ref, iy_ref, small_ref):
    gp = gp_ref[...]
    rows = lax.broadcasted_iota(jnp.int32, (V_PAD, V_PAD), 0)
    cols = lax.broadcasted_iota(jnp.int32, (V_PAD, V_PAD), 1)
    sp = jnp.maximum(gp, 0.0) + jnp.log(1.0 + jnp.exp(-jnp.abs(gp)))
    L = jnp.where(cols < rows, gp, jnp.where(cols == rows, sp, 0.0))
    gram = lax.dot_general(L, L, (((1,), (1,)), ((), ())),
                           preferred_element_type=jnp.float32)
    iy = iy_ref[0, :]
    onehot = (rows == iy[None, :]).astype(jnp.float32)
    small_ref[...] = jnp.dot(gram, onehot, preferred_element_type=jnp.float32)


_tc_table = pl.pallas_call(
    _tc_table_body,
    out_shape=jax.ShapeDtypeStruct((V_PAD, N_Y), jnp.float32),
)


@functools.cache
def _make_sc_gather():
    mesh = plsc.VectorSubcoreMesh(core_axis_name="c", subcore_axis_name="s")

    @functools.partial(
        pl.kernel,
        mesh=mesh,
        out_type=jax.ShapeDtypeStruct((N_X, N_Y), jnp.float32),
        scratch_types=[
            pltpu.VMEM((_BPW,), jnp.int32),
            pltpu.VMEM((_CHUNK, N_Y), jnp.float32),
            pltpu.SemaphoreType.DMA,
        ],
    )
    def _sc_gather(table_hbm, idx_hbm, out_hbm, idx_v, rows_v, sem):
        wid = lax.axis_index("s") * _NC + lax.axis_index("c")
        base = wid * _BPW
        pltpu.sync_copy(idx_hbm.at[pl.ds(base, _BPW)], idx_v)
        for c in range(_NCHUNK):
            pltpu.async_copy(
                table_hbm.at[idx_v.at[pl.ds(c * _CHUNK, _CHUNK)]], rows_v, sem
            ).wait()
            pltpu.sync_copy(rows_v, out_hbm.at[pl.ds(base + c * _CHUNK, _CHUNK)])

    return _sc_gather


def kernel(gram_param, idx_X, idx_Y):
    v = gram_param.shape[0]
    gp = jnp.pad(gram_param, ((0, V_PAD - v), (0, V_PAD - v)))
    iy = idx_Y.reshape(1, -1).astype(jnp.int32)
    small = _tc_table(gp, iy)
    ix = idx_X.reshape(-1).astype(jnp.int32)
    return _make_sc_gather()(small, ix)


# speedup vs baseline: 2508.4318x; 1.0367x over previous
import functools

import jax
import jax.numpy as jnp
from jax import lax
from jax.experimental import pallas as pl
from jax.experimental.pallas import tpu as pltpu
from jax.experimental.pallas import tpu_sc as plsc

V_PAD = 1024
N_X = 16384
N_Y = 1024

_NC, _NS = 2, 16
_NW = _NC * _NS
_BPW = N_X // _NW
_CHUNK = 32
_NCHUNK = _BPW // _CHUNK


def _tc_table_body(gp_ref, iy_ref, small_ref):
    gp = gp_ref[...]
    rows = lax.broadcasted_iota(jnp.int32, (V_PAD, V_PAD), 0)
    cols = lax.broadcasted_iota(jnp.int32, (V_PAD, V_PAD), 1)
    sp = jnp.maximum(gp, 0.0) + jnp.log(1.0 + jnp.exp(-jnp.abs(gp)))
    L = jnp.where(cols < rows, gp, jnp.where(cols == rows, sp, 0.0))
    gram = lax.dot_general(L, L, (((1,), (1,)), ((), ())),
                           preferred_element_type=jnp.float32)
    iy = iy_ref[0, :]
    onehot = (rows == iy[None, :]).astype(jnp.float32)
    small_ref[...] = jnp.dot(gram, onehot, preferred_element_type=jnp.float32)


_tc_table = pl.pallas_call(
    _tc_table_body,
    out_shape=jax.ShapeDtypeStruct((V_PAD, N_Y), jnp.float32),
)


@functools.cache
def _make_sc_gather():
    mesh = plsc.VectorSubcoreMesh(core_axis_name="c", subcore_axis_name="s")

    @functools.partial(
        pl.kernel,
        mesh=mesh,
        out_type=jax.ShapeDtypeStruct((N_X, N_Y), jnp.float32),
        scratch_types=[
            pltpu.VMEM((_BPW,), jnp.int32),
            pltpu.VMEM((_CHUNK, N_Y), jnp.float32),
            pltpu.VMEM((_CHUNK, N_Y), jnp.float32),
            pltpu.SemaphoreType.DMA,
            pltpu.SemaphoreType.DMA,
            pltpu.SemaphoreType.DMA,
            pltpu.SemaphoreType.DMA,
        ],
    )
    def _sc_gather(table_hbm, idx_hbm, out_hbm, idx_v, b0, b1, g0, g1, s0, s1):
        wid = lax.axis_index("s") * _NC + lax.axis_index("c")
        base = wid * _BPW
        pltpu.sync_copy(idx_hbm.at[pl.ds(base, _BPW)], idx_v)
        bufs, gsem, ssem = (b0, b1), (g0, g1), (s0, s1)

        def start_gather(c):
            b = c & 1
            return pltpu.async_copy(
                table_hbm.at[idx_v.at[pl.ds(c * _CHUNK, _CHUNK)]], bufs[b], gsem[b]
            )

        gathers = {0: start_gather(0), 1: start_gather(1)}
        scatters = {}
        for c in range(_NCHUNK):
            b = c & 1
            gathers[c].wait()
            scatters[c] = pltpu.async_copy(
                bufs[b], out_hbm.at[pl.ds(base + c * _CHUNK, _CHUNK)], ssem[b]
            )
            if c + 2 < _NCHUNK:
                scatters[c].wait()
                gathers[c + 2] = start_gather(c + 2)
        scatters[_NCHUNK - 2].wait()
        scatters[_NCHUNK - 1].wait()

    return _sc_gather


def kernel(gram_param, idx_X, idx_Y):
    v = gram_param.shape[0]
    gp = jnp.pad(gram_param, ((0, V_PAD - v), (0, V_PAD - v)))
    iy = idx_Y.reshape(1, -1).astype(jnp.int32)
    small = _tc_table(gp, iy)
    ix = idx_X.reshape(-1).astype(jnp.int32)
    return _make_sc_gather()(small, ix)


# speedup vs baseline: 2673.8017x; 1.0659x over previous
import functools

import jax
import jax.numpy as jnp
from jax import lax
from jax.experimental import pallas as pl
from jax.experimental.pallas import tpu as pltpu
from jax.experimental.pallas import tpu_sc as plsc

V_PAD = 1024
_V_REAL = 1000
N_X = 16384
N_Y = 1024

_NC, _NS = 2, 16
_NW = _NC * _NS
_BPW = N_X // _NW
_CHUNK = 32
_NCHUNK = _BPW // _CHUNK


def _tc_table_body(gp_ref, iy_ref, small_ref):
    gp = gp_ref[...]
    rows = lax.broadcasted_iota(jnp.int32, (V_PAD, V_PAD), 0)
    cols = lax.broadcasted_iota(jnp.int32, (V_PAD, V_PAD), 1)
    sp = jnp.maximum(gp, 0.0) + jnp.log(1.0 + jnp.exp(-jnp.abs(gp)))
    valid = (rows < _V_REAL) & (cols < _V_REAL)
    L = jnp.where(valid & (cols < rows), gp,
                  jnp.where(valid & (cols == rows), sp, 0.0))
    gram = lax.dot_general(L, L, (((1,), (1,)), ((), ())),
                           preferred_element_type=jnp.float32)
    iy = iy_ref[0, :]
    onehot = (rows == iy[None, :]).astype(jnp.float32)
    small_ref[...] = jnp.dot(gram, onehot, preferred_element_type=jnp.float32)


_tc_table = pl.pallas_call(
    _tc_table_body,
    grid=(1,),
    in_specs=[
        pl.BlockSpec((V_PAD, V_PAD), lambda i: (0, 0)),
        pl.BlockSpec((1, N_Y), lambda i: (0, 0)),
    ],
    out_specs=pl.BlockSpec((V_PAD, N_Y), lambda i: (0, 0)),
    out_shape=jax.ShapeDtypeStruct((V_PAD, N_Y), jnp.float32),
)


@functools.cache
def _make_sc_gather():
    mesh = plsc.VectorSubcoreMesh(core_axis_name="c", subcore_axis_name="s")

    @functools.partial(
        pl.kernel,
        mesh=mesh,
        out_type=jax.ShapeDtypeStruct((N_X, N_Y), jnp.float32),
        scratch_types=[
            pltpu.VMEM((_BPW,), jnp.int32),
            pltpu.VMEM((_CHUNK, N_Y), jnp.float32),
            pltpu.VMEM((_CHUNK, N_Y), jnp.float32),
            pltpu.SemaphoreType.DMA,
            pltpu.SemaphoreType.DMA,
            pltpu.SemaphoreType.DMA,
            pltpu.SemaphoreType.DMA,
        ],
    )
    def _sc_gather(table_hbm, idx_hbm, out_hbm, idx_v, b0, b1, g0, g1, s0, s1):
        sid = lax.axis_index("s")
        wid = sid * _NC + lax.axis_index("c")
        base = wid * _BPW
        pltpu.sync_copy(idx_hbm.at[pl.ds(base, _BPW)], idx_v)
        bufs, gsem, ssem = (b0, b1), (g0, g1), (s0, s1)

        def start_gather(c):
            b = c & 1
            return pltpu.async_copy(
                table_hbm.at[idx_v.at[pl.ds(c * _CHUNK, _CHUNK)]], bufs[b], gsem[b]
            )

        gathers = {0: start_gather(0), 1: start_gather(1)}
        scatters = {}
        for c in range(_NCHUNK):
            b = c & 1
            gathers[c].wait()
            scatters[c] = pltpu.async_copy(
                bufs[b], out_hbm.at[pl.ds(base + c * _CHUNK, _CHUNK)], ssem[b]
            )
            if c + 2 < _NCHUNK:
                scatters[c].wait()
                gathers[c + 2] = start_gather(c + 2)
        scatters[_NCHUNK - 2].wait()
        scatters[_NCHUNK - 1].wait()

    return _sc_gather


def kernel(gram_param, idx_X, idx_Y):
    iy = idx_Y.reshape(1, -1).astype(jnp.int32)
    small = _tc_table(gram_param, iy)
    ix = idx_X.reshape(-1).astype(jnp.int32)
    return _make_sc_gather()(small, ix)
